# baseline (speedup 1.0000x reference)
import functools

import numpy as np
import jax
import jax.numpy as jnp
from jax import lax
from jax.experimental import pallas as pl
from jax.experimental.pallas import tpu as pltpu
from jax.experimental.pallas import tpu_sc as plsc

def _fori32(n, body):
    def step(_, k):
        body(k)
        return k + np.int32(1)

    lax.fori_loop(0, n, step, np.int32(0))


CAP = 256
NELEM = 128
ROW = 100 * 3 * 32 * 32
CHUNK = 51200
CPR = ROW // CHUNK
NWORK = 32
RPW = CAP // NWORK


def _sc_row_mover():
    mesh = plsc.VectorSubcoreMesh(core_axis_name="c", subcore_axis_name="s")

    @functools.partial(
        pl.kernel,
        mesh=mesh,
        compiler_params=pltpu.CompilerParams(needs_layout_passes=False),
        out_type=jax.ShapeDtypeStruct((CAP * CPR, CHUNK), jnp.float32),
        scratch_types=[
            pltpu.VMEM((NELEM,), jnp.int32),
            pltpu.VMEM((1, CHUNK), jnp.float32),
            pltpu.VMEM((1, CHUNK), jnp.float32),
            pltpu.SemaphoreType.DMA,
            pltpu.SemaphoreType.DMA,
            pltpu.SemaphoreType.DMA,
            pltpu.SemaphoreType.DMA,
        ],
    )
    def mover(x_hbm, idx_hbm, bx_hbm, out_hbm, idx_v, buf0, buf1,
              si0, si1, so0, so1):
        wid = lax.axis_index("c") * 16 + lax.axis_index("s")
        pltpu.sync_copy(idx_hbm, idx_v)
        iota = lax.iota(jnp.int32, 16)

        def copy_row(src_hbm, sbase, dbase):
            def pair(tt):
                r = sbase + 2 * tt
                d = dbase + 2 * tt
                in0 = pltpu.make_async_copy(
                    src_hbm.at[pl.ds(r, 1)], buf0, si0)
                in0.start()
                in1 = pltpu.make_async_copy(
                    src_hbm.at[pl.ds(r + 1, 1)], buf1, si1)
                in1.start()
                in0.wait()
                out0 = pltpu.make_async_copy(
                    buf0, out_hbm.at[pl.ds(d, 1)], so0)
                out0.start()
                in1.wait()
                out1 = pltpu.make_async_copy(
                    buf1, out_hbm.at[pl.ds(d + 1, 1)], so1)
                out1.start()
                out0.wait()
                out1.wait()
            _fori32(CPR // 2, pair)

        def row_body(k):
            c = wid * RPW + k
            acc = jnp.full((16,), -1, jnp.int32)
            for m in range(NELEM // 16):
                chunk = idx_v[pl.ds(16 * m, 16)]
                jv = iota + (16 * m)
                acc = jnp.maximum(acc, jnp.where(chunk == c, jv, -1))
            s = jnp.max(acc)
            dbase = c * CPR

            @pl.when(s >= 0)
            def _():
                copy_row(x_hbm, s * CPR, dbase)

            @pl.when(s < 0)
            def _():
                copy_row(bx_hbm, dbase, dbase)

        _fori32(RPW, row_body)

    return mover


_SMALL_N = 2 * CAP


def _tc_small_body(idx_ref, y_ref, t_ref, by_ref, bt_ref, nby_ref, nbt_ref):
    def cp(i):
        nby_ref[i] = by_ref[i]
        nbt_ref[i] = bt_ref[i]
    _fori32(_SMALL_N, cp)

    def scat(j):
        i = idx_ref[j]

        @pl.when(i < CAP)
        def _():
            nby_ref[2 * i] = y_ref[2 * j]
            nby_ref[2 * i + 1] = y_ref[2 * j + 1]
            nbt_ref[2 * i] = t_ref[2 * j]
            nbt_ref[2 * i + 1] = t_ref[2 * j + 1]

    _fori32(NELEM, scat)


_tc_small = pl.pallas_call(
    _tc_small_body,
    out_shape=(jax.ShapeDtypeStruct((_SMALL_N,), jnp.int32),
               jax.ShapeDtypeStruct((_SMALL_N,), jnp.int32)),
    in_specs=[pl.BlockSpec(memory_space=pltpu.SMEM)] * 5,
    out_specs=(pl.BlockSpec(memory_space=pltpu.SMEM),
               pl.BlockSpec(memory_space=pltpu.SMEM)),
)


def kernel(x, y, t, indices, bx, by, bt):
    xr = x.reshape(NELEM * CPR, CHUNK)
    bxr = bx.reshape(CAP * CPR, CHUNK)
    idx32 = indices.astype(jnp.int32)

    outr = _sc_row_mover()(xr, idx32, bxr)
    new_bx = outr.reshape(bx.shape)

    y32 = lax.bitcast_convert_type(y, jnp.int32).reshape(2 * NELEM)
    t32 = lax.bitcast_convert_type(t, jnp.int32).reshape(2 * NELEM)
    by32 = lax.bitcast_convert_type(by, jnp.int32).reshape(_SMALL_N)
    bt32 = lax.bitcast_convert_type(bt, jnp.int32).reshape(_SMALL_N)

    nby32, nbt32 = _tc_small(idx32, y32, t32, by32, bt32)
    new_by = lax.bitcast_convert_type(nby32.reshape(CAP, 2), jnp.int64)
    new_bt = lax.bitcast_convert_type(nbt32.reshape(CAP, 2), jnp.int64)
    return new_bx, new_by, new_bt


# speedup vs baseline: 1.3934x; 1.3934x over previous
import functools

import numpy as np
import jax
import jax.numpy as jnp
from jax import lax
from jax.experimental import pallas as pl
from jax.experimental.pallas import tpu as pltpu
from jax.experimental.pallas import tpu_sc as plsc

def _fori32(n, body):
    def step(_, k):
        body(k)
        return k + np.int32(1)

    lax.fori_loop(0, n, step, np.int32(0))


CAP = 256
NELEM = 128
ROW = 100 * 3 * 32 * 32
CHB = 61440
CPRW = ROW // CHB
NWORK = 32
RPW = CAP // NWORK


def _sc_row_mover():
    mesh = plsc.VectorSubcoreMesh(core_axis_name="c", subcore_axis_name="s")

    @functools.partial(
        pl.kernel,
        mesh=mesh,
        compiler_params=pltpu.CompilerParams(needs_layout_passes=False),
        out_type=jax.ShapeDtypeStruct((CAP * ROW,), jnp.float32),
        scratch_types=[
            pltpu.VMEM((NELEM,), jnp.int32),
            pltpu.VMEM((CHB,), jnp.float32),
            pltpu.VMEM((CHB,), jnp.float32),
            pltpu.SemaphoreType.DMA,
            pltpu.SemaphoreType.DMA,
            pltpu.SemaphoreType.DMA,
            pltpu.SemaphoreType.DMA,
        ],
    )
    def mover(x_hbm, idx_hbm, bx_hbm, out_hbm, idx_v, buf0, buf1,
              si0, si1, so0, so1):
        wid = lax.axis_index("c") * 16 + lax.axis_index("s")
        pltpu.sync_copy(idx_hbm, idx_v)
        iota = lax.iota(jnp.int32, 16)
        bufs = (buf0, buf1)
        sin = (si0, si1)
        sout = (so0, so1)

        def copy_row(src_hbm, sbase, dbase, k):
            for t in range(CPRW):
                b = t % 2
                drain = pltpu.make_async_copy(
                    bufs[b], out_hbm.at[pl.ds(dbase, CHB)], sout[b])
                if t < 2:
                    @pl.when(k >= 1)
                    def _():
                        drain.wait()
                else:
                    drain.wait()
                cin = pltpu.make_async_copy(
                    src_hbm.at[pl.ds(sbase + t * CHB, CHB)], bufs[b], sin[b])
                cin.start()
                cin.wait()
                pltpu.make_async_copy(
                    bufs[b], out_hbm.at[pl.ds(dbase + t * CHB, CHB)],
                    sout[b]).start()

        def row_body(k):
            c = wid * RPW + k
            acc = jnp.full((16,), -1, jnp.int32)
            for m in range(NELEM // 16):
                chunk = idx_v[pl.ds(16 * m, 16)]
                jv = iota + (16 * m)
                acc = jnp.maximum(acc, jnp.where(chunk == c, jv, -1))
            s = jnp.max(acc)
            dbase = c * ROW

            @pl.when(s >= 0)
            def _():
                copy_row(x_hbm, s * ROW, dbase, k)

            @pl.when(s < 0)
            def _():
                copy_row(bx_hbm, dbase, dbase, k)

        _fori32(RPW, row_body)

        tail = wid * RPW * ROW
        for b in range(2):
            pltpu.make_async_copy(
                bufs[b], out_hbm.at[pl.ds(tail, CHB)], sout[b]).wait()

    return mover


_SMALL_N = 2 * CAP


def _tc_small_body(idx_ref, y_ref, t_ref, by_ref, bt_ref, nby_ref, nbt_ref):
    def cp(i):
        nby_ref[i] = by_ref[i]
        nbt_ref[i] = bt_ref[i]
    _fori32(_SMALL_N, cp)

    def scat(j):
        i = idx_ref[j]

        @pl.when(i < CAP)
        def _():
            nby_ref[2 * i] = y_ref[2 * j]
            nby_ref[2 * i + 1] = y_ref[2 * j + 1]
            nbt_ref[2 * i] = t_ref[2 * j]
            nbt_ref[2 * i + 1] = t_ref[2 * j + 1]

    _fori32(NELEM, scat)


_tc_small = pl.pallas_call(
    _tc_small_body,
    out_shape=(jax.ShapeDtypeStruct((_SMALL_N,), jnp.int32),
               jax.ShapeDtypeStruct((_SMALL_N,), jnp.int32)),
    in_specs=[pl.BlockSpec(memory_space=pltpu.SMEM)] * 5,
    out_specs=(pl.BlockSpec(memory_space=pltpu.SMEM),
               pl.BlockSpec(memory_space=pltpu.SMEM)),
)


def kernel(x, y, t, indices, bx, by, bt):
    xr = x.reshape(NELEM * ROW)
    bxr = bx.reshape(CAP * ROW)
    idx32 = indices.astype(jnp.int32)

    outr = _sc_row_mover()(xr, idx32, bxr)
    new_bx = outr.reshape(bx.shape)

    y32 = lax.bitcast_convert_type(y, jnp.int32).reshape(2 * NELEM)
    t32 = lax.bitcast_convert_type(t, jnp.int32).reshape(2 * NELEM)
    by32 = lax.bitcast_convert_type(by, jnp.int32).reshape(_SMALL_N)
    bt32 = lax.bitcast_convert_type(bt, jnp.int32).reshape(_SMALL_N)

    nby32, nbt32 = _tc_small(idx32, y32, t32, by32, bt32)
    new_by = lax.bitcast_convert_type(nby32.reshape(CAP, 2), jnp.int64)
    new_bt = lax.bitcast_convert_type(nbt32.reshape(CAP, 2), jnp.int64)
    return new_bx, new_by, new_bt


# speedup vs baseline: 6.1423x; 4.4083x over previous
import functools

import numpy as np
import jax
import jax.numpy as jnp
from jax import lax
from jax.experimental import pallas as pl
from jax.experimental.pallas import tpu as pltpu
from jax.experimental.pallas import tpu_sc as plsc


def _fori32(n, body):
    def step(_, k):
        body(k)
        return k + np.int32(1)

    lax.fori_loop(0, n, step, np.int32(0))


CAP = 256
NELEM = 128
PIX = 100 * 3 * 32 * 32
NWORK = 32
PPW = PIX // NWORK
P = 96
NSLAB = PPW // P
NLANE = NELEM + CAP


def _sc_lane_gather():
    mesh = plsc.VectorSubcoreMesh(core_axis_name="c", subcore_axis_name="s")

    @functools.partial(
        pl.kernel,
        mesh=mesh,
        compiler_params=pltpu.CompilerParams(needs_layout_passes=False),
        out_type=jax.ShapeDtypeStruct((PIX, CAP), jnp.float32),
        scratch_types=[
            pltpu.VMEM((NELEM,), jnp.int32),
            pltpu.VMEM((P, NLANE), jnp.float32),
            pltpu.VMEM((P, NLANE), jnp.float32),
            pltpu.VMEM((P, CAP), jnp.float32),
            pltpu.VMEM((P, CAP), jnp.float32),
            pltpu.SemaphoreType.DMA,
            pltpu.SemaphoreType.DMA,
            pltpu.SemaphoreType.DMA,
            pltpu.SemaphoreType.DMA,
            pltpu.SemaphoreType.DMA,
            pltpu.SemaphoreType.DMA,
        ],
    )
    def mover(x_hbm, idx_hbm, bx_hbm, out_hbm, idx_v, in0, in1, ob0, ob1,
              sx0, sx1, sb0, sb1, so0, so1):
        wid = lax.axis_index("c") * 16 + lax.axis_index("s")
        base_w = wid * PPW
        pltpu.sync_copy(idx_hbm, idx_v)
        iota = lax.iota(jnp.int32, 16)
        inbuf = (in0, in1)
        outbuf = (ob0, ob1)
        semx = (sx0, sx1)
        semb = (sb0, sb1)
        semo = (so0, so1)

        gvecs = []
        for g in range(16):
            cvec = iota + 16 * g
            acc = jnp.full((16,), -1, jnp.int32)
            for m in range(NELEM // 16):
                for r in range(16):
                    jv = jnp.where(iota >= 16 - r, iota + (r - 16),
                                   iota + r) + 16 * m
                    vals = plsc.load_gather(idx_v, [jv])
                    acc = jnp.maximum(acc,
                                      jnp.where(vals == cvec, jv, -1))
            gvecs.append(jnp.where(acc >= 0, acc, cvec + NELEM))

        def in_start(i, b):
            base = base_w + i * P
            pltpu.make_async_copy(
                x_hbm.at[pl.ds(base, P)],
                inbuf[b].at[:, pl.ds(0, NELEM)], semx[b]).start()
            pltpu.make_async_copy(
                bx_hbm.at[pl.ds(base, P)],
                inbuf[b].at[:, pl.ds(NELEM, CAP)], semb[b]).start()

        def in_wait(b):
            pltpu.make_async_copy(
                x_hbm.at[pl.ds(base_w, P)],
                inbuf[b].at[:, pl.ds(0, NELEM)], semx[b]).wait()
            pltpu.make_async_copy(
                bx_hbm.at[pl.ds(base_w, P)],
                inbuf[b].at[:, pl.ds(NELEM, CAP)], semb[b]).wait()

        def out_wait(b):
            pltpu.make_async_copy(
                outbuf[b], out_hbm.at[pl.ds(base_w, P)], semo[b]).wait()

        in_start(np.int32(0), 0)

        def super_body(i2):
            for b in range(2):
                i = 2 * i2 + b
                in_wait(b)

                @pl.when(i + 1 < NSLAB)
                def _():
                    in_start(i + 1, 1 - b)

                @pl.when(i >= 2)
                def _():
                    out_wait(b)

                def px_body(p):
                    pp = jnp.full((16,), 0, jnp.int32) + p
                    for g in range(16):
                        vals = plsc.load_gather(inbuf[b], [pp, gvecs[g]])
                        outbuf[b][p, pl.ds(16 * g, 16)] = vals
                _fori32(P, px_body)

                pltpu.make_async_copy(
                    outbuf[b],
                    out_hbm.at[pl.ds(base_w + i * P, P)], semo[b]).start()

        _fori32(NSLAB // 2, super_body)
        out_wait(0)
        out_wait(1)

    return mover


_SMALL_N = 2 * CAP


def _tc_small_body(idx_ref, y_ref, t_ref, by_ref, bt_ref, nby_ref, nbt_ref):
    def cp(i):
        nby_ref[i] = by_ref[i]
        nbt_ref[i] = bt_ref[i]
    _fori32(_SMALL_N, cp)

    def scat(j):
        i = idx_ref[j]

        @pl.when(i < CAP)
        def _():
            nby_ref[2 * i] = y_ref[2 * j]
            nby_ref[2 * i + 1] = y_ref[2 * j + 1]
            nbt_ref[2 * i] = t_ref[2 * j]
            nbt_ref[2 * i + 1] = t_ref[2 * j + 1]

    _fori32(NELEM, scat)


_tc_small = pl.pallas_call(
    _tc_small_body,
    out_shape=(jax.ShapeDtypeStruct((_SMALL_N,), jnp.int32),
               jax.ShapeDtypeStruct((_SMALL_N,), jnp.int32)),
    in_specs=[pl.BlockSpec(memory_space=pltpu.SMEM)] * 5,
    out_specs=(pl.BlockSpec(memory_space=pltpu.SMEM),
               pl.BlockSpec(memory_space=pltpu.SMEM)),
)


def kernel(x, y, t, indices, bx, by, bt):
    xt = jnp.transpose(x, (1, 2, 3, 4, 0)).reshape(PIX, NELEM)
    bxt = jnp.transpose(bx, (1, 2, 3, 4, 0)).reshape(PIX, CAP)
    idx32 = indices.astype(jnp.int32)

    outt = _sc_lane_gather()(xt, idx32, bxt)
    new_bx = jnp.transpose(
        outt.reshape(100, 3, 32, 32, CAP), (4, 0, 1, 2, 3))

    y32 = lax.bitcast_convert_type(y, jnp.int32).reshape(2 * NELEM)
    t32 = lax.bitcast_convert_type(t, jnp.int32).reshape(2 * NELEM)
    by32 = lax.bitcast_convert_type(by, jnp.int32).reshape(_SMALL_N)
    bt32 = lax.bitcast_convert_type(bt, jnp.int32).reshape(_SMALL_N)

    nby32, nbt32 = _tc_small(idx32, y32, t32, by32, bt32)
    new_by = lax.bitcast_convert_type(nby32.reshape(CAP, 2), jnp.int64)
    new_bt = lax.bitcast_convert_type(nbt32.reshape(CAP, 2), jnp.int64)
    return new_bx, new_by, new_bt


# speedup vs baseline: 6.5874x; 1.0725x over previous
import functools

import numpy as np
import jax
import jax.numpy as jnp
from jax import lax
from jax.experimental import pallas as pl
from jax.experimental.pallas import tpu as pltpu
from jax.experimental.pallas import tpu_sc as plsc


def _fori32(n, body):
    def step(_, k):
        body(k)
        return k + np.int32(1)

    lax.fori_loop(0, n, step, np.int32(0))


CAP = 256
NELEM = 128
PIX = 100 * 3 * 32 * 32
NWORK = 32
PPW = PIX // NWORK
P = 160
NSLAB = PPW // P
WX = NELEM
WO = CAP


def _sc_lane_gather():
    mesh = plsc.VectorSubcoreMesh(core_axis_name="c", subcore_axis_name="s")

    @functools.partial(
        pl.kernel,
        mesh=mesh,
        compiler_params=pltpu.CompilerParams(needs_layout_passes=False),
        out_type=jax.ShapeDtypeStruct((PIX, CAP), jnp.float32),
        scratch_types=[
            pltpu.VMEM((NELEM,), jnp.int32),
            pltpu.VMEM((P, WX), jnp.float32),
            pltpu.VMEM((P, WX), jnp.float32),
            pltpu.VMEM((P, WO), jnp.float32),
            pltpu.VMEM((P, WO), jnp.float32),
            pltpu.SMEM((NELEM,), jnp.int32),
            pltpu.SMEM((NELEM,), jnp.int32),
            pltpu.SemaphoreType.DMA,
            pltpu.SemaphoreType.DMA,
            pltpu.SemaphoreType.DMA,
            pltpu.SemaphoreType.DMA,
            pltpu.SemaphoreType.DMA,
            pltpu.SemaphoreType.DMA,
        ],
    )
    def mover(x_hbm, idx_hbm, bx_hbm, out_hbm, idx_v, in0, in1, ob0, ob1,
              scat_c, scat_s, sx0, sx1, sb0, sb1, so0, so1):
        wid = lax.axis_index("c") * 16 + lax.axis_index("s")
        base_w = wid * PPW
        pltpu.sync_copy(idx_hbm, idx_v)
        iota = lax.iota(jnp.int32, 16)
        inbuf = (in0, in1)
        outbuf = (ob0, ob1)
        semx = (sx0, sx1)
        semb = (sb0, sb1)
        semo = (so0, so1)

        def grp_body(_, carry):
            g, off = carry
            cvec = iota + g * np.int32(16)
            acc = jnp.full((16,), -1, jnp.int32)

            def m_body(_, mcarry):
                m, a = mcarry
                for r in range(16):
                    jv = jnp.where(iota >= 16 - r, iota + (r - 16),
                                   iota + r) + 16 * m
                    vals = plsc.load_gather(idx_v, [jv])
                    a = jnp.maximum(a, jnp.where(vals == cvec, jv, -1))
                return m + np.int32(1), a

            _, acc = lax.fori_loop(0, NELEM // 16, m_body,
                                   (np.int32(0), acc))

            def l_body(_, lcarry):
                lo, loff = lcarry
                s = jnp.max(jnp.where(iota == lo, acc,
                                      np.int32(-(2**20))))

                @pl.when(s >= 0)
                def _():
                    scat_c[loff] = g * np.int32(16) + lo
                    scat_s[loff] = s

                return (lo + np.int32(1),
                        jnp.where(s >= 0, loff + np.int32(1), loff))

            _, off = lax.fori_loop(0, 16, l_body, (np.int32(0), off))
            return g + np.int32(1), off

        _, nscat = lax.fori_loop(0, CAP // 16, grp_body,
                                 (np.int32(0), np.int32(0)))

        zero16 = jnp.full((16,), 0, jnp.int32)

        @pl.when(nscat > 0)
        def _():
            def fill(k):
                @pl.when(k >= nscat)
                def _():
                    scat_c[k] = scat_c[0]
                    scat_s[k] = scat_s[0]
            _fori32(NELEM, fill)

        svecs = []
        cvecs = []
        for q in range(8):
            sv = zero16
            cv = zero16
            for l in range(16):
                sv = jnp.where(iota == l, zero16 + scat_s[16 * q + l], sv)
                cv = jnp.where(iota == l, zero16 + scat_c[16 * q + l], cv)
            svecs.append(sv)
            cvecs.append(cv)

        def start_x(i, b):
            pltpu.make_async_copy(
                x_hbm.at[pl.ds(base_w + i * P, P)],
                inbuf[b].at[:, pl.ds(0, NELEM)], semx[b]).start()

        def start_bx(i, b):
            pltpu.make_async_copy(
                bx_hbm.at[pl.ds(base_w + i * P, P)],
                outbuf[b].at[:, pl.ds(0, CAP)], semb[b]).start()

        def start_out(i, b):
            pltpu.make_async_copy(
                outbuf[b].at[:, pl.ds(0, CAP)],
                out_hbm.at[pl.ds(base_w + i * P, P)], semo[b]).start()

        def wait_x(b):
            pltpu.make_async_copy(
                x_hbm.at[pl.ds(base_w, P)],
                inbuf[b].at[:, pl.ds(0, NELEM)], semx[b]).wait()

        def wait_bx(b):
            pltpu.make_async_copy(
                bx_hbm.at[pl.ds(base_w, P)],
                outbuf[b].at[:, pl.ds(0, CAP)], semb[b]).wait()

        def wait_out(b):
            pltpu.make_async_copy(
                outbuf[b].at[:, pl.ds(0, CAP)],
                out_hbm.at[pl.ds(base_w, P)], semo[b]).wait()

        def compute(b):
            def px_body(p):
                pp = zero16 + p
                for q in range(8):
                    @pl.when(16 * q < nscat)
                    def _():
                        vals = plsc.load_gather(inbuf[b], [pp, svecs[q]])
                        plsc.store_scatter(outbuf[b], [pp, cvecs[q]],
                                           vals)
            _fori32(P, px_body)

        start_x(np.int32(0), 0)
        start_bx(np.int32(0), 0)
        start_x(np.int32(1), 1)
        start_bx(np.int32(1), 1)

        def super_body(i2):
            for b in range(2):
                j = 2 * i2 + b
                wait_x(b)
                wait_bx(b)
                compute(b)
                start_out(j, b)

                @pl.when(j + 2 < NSLAB)
                def _():
                    start_x(j + 2, b)

                @pl.when(jnp.logical_and(j >= 1, j + 1 < NSLAB))
                def _():
                    wait_out(1 - b)
                    start_bx(j + 1, 1 - b)

        _fori32(NSLAB // 2, super_body)
        wait_out(0)
        wait_out(1)

    return mover


_SMALL_N = 2 * CAP


def _tc_small_body(idx_ref, y_ref, t_ref, by_ref, bt_ref, nby_ref, nbt_ref):
    def cp(i):
        nby_ref[i] = by_ref[i]
        nbt_ref[i] = bt_ref[i]
    _fori32(_SMALL_N, cp)

    def scat(j):
        i = idx_ref[j]

        @pl.when(i < CAP)
        def _():
            nby_ref[2 * i] = y_ref[2 * j]
            nby_ref[2 * i + 1] = y_ref[2 * j + 1]
            nbt_ref[2 * i] = t_ref[2 * j]
            nbt_ref[2 * i + 1] = t_ref[2 * j + 1]

    _fori32(NELEM, scat)


_tc_small = pl.pallas_call(
    _tc_small_body,
    out_shape=(jax.ShapeDtypeStruct((_SMALL_N,), jnp.int32),
               jax.ShapeDtypeStruct((_SMALL_N,), jnp.int32)),
    in_specs=[pl.BlockSpec(memory_space=pltpu.SMEM)] * 5,
    out_specs=(pl.BlockSpec(memory_space=pltpu.SMEM),
               pl.BlockSpec(memory_space=pltpu.SMEM)),
)


def kernel(x, y, t, indices, bx, by, bt):
    xt = jnp.transpose(x, (1, 2, 3, 4, 0)).reshape(PIX, NELEM)
    bxt = jnp.transpose(bx, (1, 2, 3, 4, 0)).reshape(PIX, CAP)
    idx32 = indices.astype(jnp.int32)

    outt = _sc_lane_gather()(xt, idx32, bxt)
    new_bx = jnp.transpose(
        outt.reshape(100, 3, 32, 32, CAP), (4, 0, 1, 2, 3))

    y32 = lax.bitcast_convert_type(y, jnp.int32).reshape(2 * NELEM)
    t32 = lax.bitcast_convert_type(t, jnp.int32).reshape(2 * NELEM)
    by32 = lax.bitcast_convert_type(by, jnp.int32).reshape(_SMALL_N)
    bt32 = lax.bitcast_convert_type(bt, jnp.int32).reshape(_SMALL_N)

    nby32, nbt32 = _tc_small(idx32, y32, t32, by32, bt32)
    new_by = lax.bitcast_convert_type(nby32.reshape(CAP, 2), jnp.int64)
    new_bt = lax.bitcast_convert_type(nbt32.reshape(CAP, 2), jnp.int64)
    return new_bx, new_by, new_bt


# speedup vs baseline: 13.7027x; 2.0801x over previous
import functools

import numpy as np
import jax
import jax.numpy as jnp
from jax import lax
from jax.experimental import pallas as pl
from jax.experimental.pallas import tpu as pltpu
from jax.experimental.pallas import tpu_sc as plsc


def _fori32(n, body):
    def step(_, k):
        body(k)
        return k + np.int32(1)

    lax.fori_loop(0, n, step, np.int32(0))


CAP = 256
NELEM = 128
PIX = 100 * 3 * 32 * 32
NWORK = 32
PPW = PIX // NWORK
P = 80
NSLAB = PPW // P
WX = NELEM
WO = CAP


def _sc_lane_gather():
    mesh = plsc.VectorSubcoreMesh(core_axis_name="c", subcore_axis_name="s")

    @functools.partial(
        pl.kernel,
        mesh=mesh,
        compiler_params=pltpu.CompilerParams(needs_layout_passes=False),
        out_type=jax.ShapeDtypeStruct((PIX, CAP), jnp.float32),
        scratch_types=[
            pltpu.VMEM((NELEM,), jnp.int32),
            pltpu.VMEM((P, WX), jnp.float32),
            pltpu.VMEM((P, WX), jnp.float32),
            pltpu.VMEM((P, WO), jnp.float32),
            pltpu.VMEM((P, WO), jnp.float32),
            pltpu.VMEM((P, WO), jnp.float32),
            pltpu.VMEM((P, WO), jnp.float32),
            pltpu.SMEM((NELEM,), jnp.int32),
            pltpu.SMEM((NELEM,), jnp.int32),
            pltpu.SemaphoreType.DMA,
            pltpu.SemaphoreType.DMA,
            pltpu.SemaphoreType.DMA,
            pltpu.SemaphoreType.DMA,
            pltpu.SemaphoreType.DMA,
            pltpu.SemaphoreType.DMA,
            pltpu.SemaphoreType.DMA,
            pltpu.SemaphoreType.DMA,
            pltpu.SemaphoreType.DMA,
            pltpu.SemaphoreType.DMA,
        ],
    )
    def mover(x_hbm, idx_hbm, bx_hbm, out_hbm, idx_v, in0, in1,
              ob0, ob1, ob2, ob3, scat_c, scat_s,
              sx0, sx1, sb0, sb1, sb2, sb3, so0, so1, so2, so3):
        wid = lax.axis_index("c") * 16 + lax.axis_index("s")
        base_w = wid * PPW
        pltpu.sync_copy(idx_hbm, idx_v)
        iota = lax.iota(jnp.int32, 16)
        inbuf = (in0, in1)
        outbuf = (ob0, ob1, ob2, ob3)
        semx = (sx0, sx1)
        semb = (sb0, sb1, sb2, sb3)
        semo = (so0, so1, so2, so3)

        def grp_body(_, carry):
            g, off = carry
            cvec = iota + g * np.int32(16)
            acc = jnp.full((16,), -1, jnp.int32)

            def m_body(_, mcarry):
                m, a = mcarry
                for r in range(16):
                    jv = jnp.where(iota >= 16 - r, iota + (r - 16),
                                   iota + r) + 16 * m
                    vals = plsc.load_gather(idx_v, [jv])
                    a = jnp.maximum(a, jnp.where(vals == cvec, jv, -1))
                return m + np.int32(1), a

            _, acc = lax.fori_loop(0, NELEM // 16, m_body,
                                   (np.int32(0), acc))

            def l_body(_, lcarry):
                lo, loff = lcarry
                s = jnp.max(jnp.where(iota == lo, acc,
                                      np.int32(-(2**20))))

                @pl.when(s >= 0)
                def _():
                    scat_c[loff] = g * np.int32(16) + lo
                    scat_s[loff] = s

                return (lo + np.int32(1),
                        jnp.where(s >= 0, loff + np.int32(1), loff))

            _, off = lax.fori_loop(0, 16, l_body, (np.int32(0), off))
            return g + np.int32(1), off

        _, nscat = lax.fori_loop(0, CAP // 16, grp_body,
                                 (np.int32(0), np.int32(0)))

        zero16 = jnp.full((16,), 0, jnp.int32)

        @pl.when(nscat > 0)
        def _():
            def fill(k):
                @pl.when(k >= nscat)
                def _():
                    scat_c[k] = scat_c[0]
                    scat_s[k] = scat_s[0]
            _fori32(NELEM, fill)

        svecs = []
        cvecs = []
        for q in range(8):
            sv = zero16
            cv = zero16
            for l in range(16):
                sv = jnp.where(iota == l, zero16 + scat_s[16 * q + l], sv)
                cv = jnp.where(iota == l, zero16 + scat_c[16 * q + l], cv)
            svecs.append(sv)
            cvecs.append(cv)

        def start_x(i, b):
            pltpu.make_async_copy(
                x_hbm.at[pl.ds(base_w + i * P, P)],
                inbuf[b].at[:, pl.ds(0, NELEM)], semx[b]).start()

        def start_bx(i, b):
            pltpu.make_async_copy(
                bx_hbm.at[pl.ds(base_w + i * P, P)],
                outbuf[b].at[:, pl.ds(0, CAP)], semb[b]).start()

        def start_out(i, b):
            pltpu.make_async_copy(
                outbuf[b].at[:, pl.ds(0, CAP)],
                out_hbm.at[pl.ds(base_w + i * P, P)], semo[b]).start()

        def wait_x(b):
            pltpu.make_async_copy(
                x_hbm.at[pl.ds(base_w, P)],
                inbuf[b].at[:, pl.ds(0, NELEM)], semx[b]).wait()

        def wait_bx(b):
            pltpu.make_async_copy(
                bx_hbm.at[pl.ds(base_w, P)],
                outbuf[b].at[:, pl.ds(0, CAP)], semb[b]).wait()

        def wait_out(b):
            pltpu.make_async_copy(
                outbuf[b].at[:, pl.ds(0, CAP)],
                out_hbm.at[pl.ds(base_w, P)], semo[b]).wait()

        def compute(b2, b4):
            for q in range(8):
                @pl.when(16 * q < nscat)
                def _(q=q):
                    def px4(t):
                        for d in range(4):
                            pp = zero16 + (4 * t + d)
                            vals = plsc.load_gather(
                                inbuf[b2], [pp, svecs[q]])
                            plsc.store_scatter(
                                outbuf[b4], [pp, cvecs[q]], vals)
                    _fori32(P // 4, px4)

        start_x(np.int32(0), 0)
        start_bx(np.int32(0), 0)
        start_x(np.int32(1), 1)
        start_bx(np.int32(1), 1)

        def super_body(i4):
            for u in range(4):
                j = 4 * i4 + u
                b2 = u % 2
                b4 = u
                wait_x(b2)
                wait_bx(b4)
                compute(b2, b4)
                start_out(j, b4)

                bn = (u + 2) % 4

                @pl.when(j + 2 < NSLAB)
                def _():
                    start_x(j + 2, b2)

                    @pl.when(j >= 2)
                    def _():
                        wait_out(bn)
                    start_bx(j + 2, bn)

        _fori32(NSLAB // 4, super_body)
        for b in range(4):
            wait_out(b)

    return mover


_SMALL_N = 2 * CAP


def _tc_small_body(idx_ref, y_ref, t_ref, by_ref, bt_ref, nby_ref, nbt_ref):
    def cp(i):
        nby_ref[i] = by_ref[i]
        nbt_ref[i] = bt_ref[i]
    _fori32(_SMALL_N, cp)

    def scat(j):
        i = idx_ref[j]

        @pl.when(i < CAP)
        def _():
            nby_ref[2 * i] = y_ref[2 * j]
            nby_ref[2 * i + 1] = y_ref[2 * j + 1]
            nbt_ref[2 * i] = t_ref[2 * j]
            nbt_ref[2 * i + 1] = t_ref[2 * j + 1]

    _fori32(NELEM, scat)


_tc_small = pl.pallas_call(
    _tc_small_body,
    out_shape=(jax.ShapeDtypeStruct((_SMALL_N,), jnp.int32),
               jax.ShapeDtypeStruct((_SMALL_N,), jnp.int32)),
    in_specs=[pl.BlockSpec(memory_space=pltpu.SMEM)] * 5,
    out_specs=(pl.BlockSpec(memory_space=pltpu.SMEM),
               pl.BlockSpec(memory_space=pltpu.SMEM)),
)


def kernel(x, y, t, indices, bx, by, bt):
    xt = jnp.transpose(x, (1, 2, 3, 4, 0)).reshape(PIX, NELEM)
    bxt = jnp.transpose(bx, (1, 2, 3, 4, 0)).reshape(PIX, CAP)
    idx32 = indices.astype(jnp.int32)

    outt = _sc_lane_gather()(xt, idx32, bxt)
    new_bx = jnp.transpose(
        outt.reshape(100, 3, 32, 32, CAP), (4, 0, 1, 2, 3))

    y32 = lax.bitcast_convert_type(y, jnp.int32).reshape(2 * NELEM)
    t32 = lax.bitcast_convert_type(t, jnp.int32).reshape(2 * NELEM)
    by32 = lax.bitcast_convert_type(by, jnp.int32).reshape(_SMALL_N)
    bt32 = lax.bitcast_convert_type(bt, jnp.int32).reshape(_SMALL_N)

    nby32, nbt32 = _tc_small(idx32, y32, t32, by32, bt32)
    new_by = lax.bitcast_convert_type(nby32.reshape(CAP, 2), jnp.int64)
    new_bt = lax.bitcast_convert_type(nbt32.reshape(CAP, 2), jnp.int64)
    return new_bx, new_by, new_bt
